# baseline (speedup 1.0000x reference)
import functools

import jax
import jax.numpy as jnp
import numpy as np
from jax import lax
from jax.experimental import pallas as pl
from jax.experimental.pallas import tpu as pltpu
from jax.experimental.pallas import tpu_sc as plsc

N = 2000000
NW = 32
L = 16
VPB = 126
BLK = VPB * L
BLK7 = BLK * 7
NBLK = 31
MAIN_ROWS = NW * NBLK * BLK
TAIL_VECS = (N - MAIN_ROWS) // L

_SC0 = np.float32(111.32 * 0.772)
_SC1 = np.float32(110.574)


def _make_kernel():
    mesh = plsc.VectorSubcoreMesh(core_axis_name="c", subcore_axis_name="s")

    @functools.partial(
        pl.kernel,
        out_type=jax.ShapeDtypeStruct((N,), jnp.float32),
        mesh=mesh,
        compiler_params=pltpu.CompilerParams(needs_layout_passes=False),
        scratch_types=[
            pltpu.VMEM((2 * BLK7,), jnp.float32),
            pltpu.VMEM((2 * BLK7,), jnp.float32),
            pltpu.VMEM((2 * BLK,), jnp.float32),
            pltpu.VMEM((64,), jnp.float32),
            pltpu.VMEM((64,), jnp.float32),
            pltpu.VMEM((64,), jnp.float32),
            pltpu.VMEM((32,), jnp.float32),
            pltpu.SemaphoreType.DMA,
            pltpu.SemaphoreType.DMA,
            pltpu.SemaphoreType.DMA,
            pltpu.SemaphoreType.DMA,
        ],
    )
    def sc_kernel(xf, yf, af, mf, sf, pf, out, xbuf, ybuf, obuf, atbl, mtbl,
                  stbl, pbuf, ls0, ls1, ss0, ss1):
        wid = lax.axis_index("s") * 2 + lax.axis_index("c")

        pltpu.sync_copy(af, atbl)
        pltpu.sync_copy(mf, mtbl)
        pltpu.sync_copy(sf, stbl)
        pltpu.sync_copy(pf, pbuf)

        for t in range(4):
            sl = pl.ds(t * L, L)
            atbl[sl] = jnp.where(mtbl[sl] != 0.0, atbl[sl], 0.0)

        beta = pbuf[pl.ds(0, L)]
        sigma = pbuf[pl.ds(L, L)]
        inv2s2 = 1.0 / (2.0 * sigma * sigma)
        cnorm = beta * inv2s2 * np.float32(1.0 / np.pi)
        iota7 = lax.iota(jnp.int32, L) * 7

        lsems = (ls0, ls1)
        ssems = (ss0, ss1)

        def flat_off(b):
            return pl.multiple_of(wid * (NBLK * BLK7) + b * BLK7, BLK7)

        NSPLIT = 4
        CHUNK = BLK7 // NSPLIT

        def fire_loads(b, slot):
            off = flat_off(b)
            for c in range(NSPLIT):
                src = pl.ds(off + c * CHUNK, CHUNK)
                dst = pl.ds(slot * BLK7 + c * CHUNK, CHUNK)
                pltpu.async_copy(xf.at[src], xbuf.at[dst], lsems[slot])
                pltpu.async_copy(yf.at[src], ybuf.at[dst], lsems[slot])

        def drain_loads(slot):
            dst = pl.ds(slot * BLK7, BLK7)
            src = pl.ds(0, BLK7)
            pltpu.make_async_copy(xf.at[src], xbuf.at[dst], lsems[slot]).wait()
            pltpu.make_async_copy(yf.at[src], ybuf.at[dst], lsems[slot]).wait()

        def drain_store(slot):
            pltpu.make_async_copy(obuf.at[pl.ds(slot * BLK, BLK)],
                                  out.at[pl.ds(0, BLK)], ssems[slot]).wait()

        def compute_vec(base):
            idx = iota7 + base
            x0 = plsc.load_gather(xbuf, [idx])
            x1 = plsc.load_gather(xbuf, [idx + 1])
            x2 = plsc.load_gather(xbuf, [idx + 2])
            x3 = plsc.load_gather(xbuf, [idx + 3])
            x4 = plsc.load_gather(xbuf, [idx + 4])
            x5 = plsc.load_gather(xbuf, [idx + 5])
            x6 = plsc.load_gather(xbuf, [idx + 6])
            y0 = plsc.load_gather(ybuf, [idx])
            y1 = plsc.load_gather(ybuf, [idx + 1])
            y2 = plsc.load_gather(ybuf, [idx + 2])
            y3 = plsc.load_gather(ybuf, [idx + 3])
            y4 = plsc.load_gather(ybuf, [idx + 4])
            y5 = plsc.load_gather(ybuf, [idx + 5])
            y6 = plsc.load_gather(ybuf, [idx + 6])

            aidx = x1.astype(jnp.int32) * 8 + y1.astype(jnp.int32)
            alphas = plsc.load_gather(atbl, [aidx])
            sidx = x4.astype(jnp.int32) * 8 + y4.astype(jnp.int32)
            spl = plsc.load_gather(stbl, [sidx])

            tds = jnp.where(x0 > 0.0, x0 - y0, jnp.float32(1.0))
            dlon = (x2 - y2) * _SC0
            dlat = (x3 - y3) * _SC1
            sq = jnp.maximum(dlon * dlon + dlat * dlat, np.float32(1e-12))
            a3 = (x5 + y5 + spl) * np.float32(1e-3)
            nw2 = jnp.where(x6 == y6, sq, a3 * a3)
            itds = 1.0 / tds
            e = jnp.exp(-(beta * tds) - nw2 * inv2s2 * itds)
            return alphas * cnorm * e * itds

        def compute_block(b, slot):
            base7 = slot * BLK7

            @plsc.parallel_loop(0, VPB, step=1, unroll=8)
            def vec_body(v):
                obuf[pl.ds(slot * BLK + v * L, L)] = compute_vec(base7 + v * 112)

            row0 = pl.multiple_of(wid * (NBLK * BLK) + b * BLK, BLK)
            pltpu.async_copy(obuf.at[pl.ds(slot * BLK, BLK)],
                             out.at[pl.ds(row0, BLK)], ssems[slot])

        fire_loads(0, 0)
        fire_loads(1, 1)

        drain_loads(0)
        compute_block(0, 0)
        fire_loads(2, 0)
        drain_loads(1)
        compute_block(1, 1)
        fire_loads(3, 1)

        def pair_body(i, carry):
            b0 = 2 * i
            drain_loads(0)
            drain_store(0)
            compute_block(b0, 0)
            fire_loads(b0 + 2, 0)
            drain_loads(1)
            drain_store(1)
            compute_block(b0 + 1, 1)
            fire_loads(b0 + 3, 1)
            return carry

        lax.fori_loop(1, 14, pair_body, 0)

        drain_loads(0)
        drain_store(0)
        compute_block(28, 0)
        fire_loads(30, 0)
        drain_loads(1)
        drain_store(1)
        compute_block(29, 1)
        drain_loads(0)
        drain_store(0)
        compute_block(30, 0)
        drain_store(0)
        drain_store(1)

        @pl.when(wid < TAIL_VECS)
        def _():
            row0 = pl.multiple_of(MAIN_ROWS + wid * L, L)
            pltpu.sync_copy(xf.at[pl.ds(row0 * 7, L * 7)],
                            xbuf.at[pl.ds(0, L * 7)])
            pltpu.sync_copy(yf.at[pl.ds(row0 * 7, L * 7)],
                            ybuf.at[pl.ds(0, L * 7)])
            obuf[pl.ds(0, L)] = compute_vec(0)
            pltpu.sync_copy(obuf.at[pl.ds(0, L)], out.at[pl.ds(row0, L)])

    return sc_kernel


_KERNEL = _make_kernel()


def kernel(x, y, alpha, beta, sigma, alpha_mask, AllSPL):
    params = jnp.concatenate([
        jnp.full((L,), beta, dtype=jnp.float32),
        jnp.full((L,), sigma, dtype=jnp.float32),
    ])
    return _KERNEL(
        x.reshape(-1),
        y.reshape(-1),
        alpha.reshape(-1),
        alpha_mask.reshape(-1),
        AllSPL.reshape(-1),
        params,
    )


# speedup vs baseline: 1.0908x; 1.0908x over previous
import functools

import jax
import jax.numpy as jnp
import numpy as np
from jax import lax
from jax.experimental import pallas as pl
from jax.experimental.pallas import tpu as pltpu
from jax.experimental.pallas import tpu_sc as plsc

N = 2000000
NW = 32
L = 16
ROWW = 112
BROWS = N * 7 // ROWW
RPB = 64
VPB = RPB
BLK = RPB * L
NBLK = 61
MAIN_BROWS = NW * NBLK * RPB
MAIN_ROWS = MAIN_BROWS * L
TAIL_BROWS = BROWS - MAIN_BROWS

_SC0 = np.float32(111.32 * 0.772)
_SC1 = np.float32(110.574)


def _make_kernel():
    mesh = plsc.VectorSubcoreMesh(core_axis_name="c", subcore_axis_name="s")

    @functools.partial(
        pl.kernel,
        out_type=jax.ShapeDtypeStruct((N,), jnp.float32),
        mesh=mesh,
        compiler_params=pltpu.CompilerParams(needs_layout_passes=False),
        scratch_types=[
            pltpu.VMEM((2 * RPB, ROWW), jnp.float32),
            pltpu.VMEM((2 * RPB, ROWW), jnp.float32),
            pltpu.VMEM((2 * BLK,), jnp.float32),
            pltpu.VMEM((64,), jnp.float32),
            pltpu.VMEM((64,), jnp.float32),
            pltpu.VMEM((64,), jnp.float32),
            pltpu.VMEM((32,), jnp.float32),
            pltpu.SemaphoreType.DMA,
            pltpu.SemaphoreType.DMA,
            pltpu.SemaphoreType.DMA,
            pltpu.SemaphoreType.DMA,
        ],
    )
    def sc_kernel(xf, yf, af, mf, sf, pf, out, xbuf, ybuf, obuf, atbl, mtbl,
                  stbl, pbuf, ls0, ls1, ss0, ss1):
        wid = lax.axis_index("s") * 2 + lax.axis_index("c")

        pltpu.sync_copy(af, atbl)
        pltpu.sync_copy(mf, mtbl)
        pltpu.sync_copy(sf, stbl)
        pltpu.sync_copy(pf, pbuf)

        for t in range(4):
            sl = pl.ds(t * L, L)
            atbl[sl] = jnp.where(mtbl[sl] != 0.0, atbl[sl], 0.0)

        beta = pbuf[pl.ds(0, L)]
        sigma = pbuf[pl.ds(L, L)]
        inv2s2 = 1.0 / (2.0 * sigma * sigma)
        cnorm = beta * inv2s2 * np.float32(1.0 / np.pi)
        iota7 = lax.iota(jnp.int32, L) * 7

        lsems = (ls0, ls1)
        ssems = (ss0, ss1)

        def row_off(b):
            return pl.multiple_of(wid * (NBLK * RPB) + b * RPB, RPB)

        def out_off(b):
            return pl.multiple_of(wid * (NBLK * BLK) + b * BLK, BLK)

        def fire_loads(b, slot):
            off = row_off(b)
            src = pl.ds(off, RPB)
            dst = pl.ds(slot * RPB, RPB)
            pltpu.async_copy(xf.at[src], xbuf.at[dst], lsems[slot])
            pltpu.async_copy(yf.at[src], ybuf.at[dst], lsems[slot])

        def drain_loads(slot):
            dst = pl.ds(slot * RPB, RPB)
            src = pl.ds(0, RPB)
            pltpu.make_async_copy(xf.at[src], xbuf.at[dst], lsems[slot]).wait()
            pltpu.make_async_copy(yf.at[src], ybuf.at[dst], lsems[slot]).wait()

        def drain_store(slot):
            pltpu.make_async_copy(obuf.at[pl.ds(slot * BLK, BLK)],
                                  out.at[pl.ds(0, BLK)], ssems[slot]).wait()

        def compute_vec(row):
            ridx = iota7 * 0 + row
            x0 = plsc.load_gather(xbuf, [ridx, iota7])
            x1 = plsc.load_gather(xbuf, [ridx, iota7 + 1])
            x2 = plsc.load_gather(xbuf, [ridx, iota7 + 2])
            x3 = plsc.load_gather(xbuf, [ridx, iota7 + 3])
            x4 = plsc.load_gather(xbuf, [ridx, iota7 + 4])
            x5 = plsc.load_gather(xbuf, [ridx, iota7 + 5])
            x6 = plsc.load_gather(xbuf, [ridx, iota7 + 6])
            y0 = plsc.load_gather(ybuf, [ridx, iota7])
            y1 = plsc.load_gather(ybuf, [ridx, iota7 + 1])
            y2 = plsc.load_gather(ybuf, [ridx, iota7 + 2])
            y3 = plsc.load_gather(ybuf, [ridx, iota7 + 3])
            y4 = plsc.load_gather(ybuf, [ridx, iota7 + 4])
            y5 = plsc.load_gather(ybuf, [ridx, iota7 + 5])
            y6 = plsc.load_gather(ybuf, [ridx, iota7 + 6])

            aidx = x1.astype(jnp.int32) * 8 + y1.astype(jnp.int32)
            alphas = plsc.load_gather(atbl, [aidx])
            sidx = x4.astype(jnp.int32) * 8 + y4.astype(jnp.int32)
            spl = plsc.load_gather(stbl, [sidx])

            tds = jnp.where(x0 > 0.0, x0 - y0, jnp.float32(1.0))
            dlon = (x2 - y2) * _SC0
            dlat = (x3 - y3) * _SC1
            sq = jnp.maximum(dlon * dlon + dlat * dlat, np.float32(1e-12))
            a3 = (x5 + y5 + spl) * np.float32(1e-3)
            nw2 = jnp.where(x6 == y6, sq, a3 * a3)
            itds = 1.0 / tds
            e = jnp.exp(-(beta * tds) - nw2 * inv2s2 * itds)
            return alphas * cnorm * e * itds

        def compute_block(b, slot):
            row0 = slot * RPB

            @plsc.parallel_loop(0, VPB, step=1, unroll=8)
            def vec_body(v):
                obuf[pl.ds(slot * BLK + v * L, L)] = compute_vec(row0 + v)

            pltpu.async_copy(obuf.at[pl.ds(slot * BLK, BLK)],
                             out.at[pl.ds(out_off(b), BLK)], ssems[slot])

        fire_loads(0, 0)
        fire_loads(1, 1)

        drain_loads(0)
        compute_block(0, 0)
        fire_loads(2, 0)
        drain_loads(1)
        compute_block(1, 1)
        fire_loads(3, 1)

        def pair_body(i, carry):
            b0 = 2 * i
            drain_loads(0)
            drain_store(0)
            compute_block(b0, 0)
            fire_loads(b0 + 2, 0)
            drain_loads(1)
            drain_store(1)
            compute_block(b0 + 1, 1)
            fire_loads(b0 + 3, 1)
            return carry

        lax.fori_loop(1, NBLK // 2 - 1, pair_body, 0)

        drain_loads(0)
        drain_store(0)
        compute_block(NBLK - 3, 0)
        fire_loads(NBLK - 1, 0)
        drain_loads(1)
        drain_store(1)
        compute_block(NBLK - 2, 1)
        drain_loads(0)
        drain_store(0)
        compute_block(NBLK - 1, 0)
        drain_store(0)
        drain_store(1)

        @pl.when(wid < 8)
        def _():
            toff = pl.multiple_of(MAIN_BROWS, 8)
            pltpu.sync_copy(xf.at[pl.ds(toff, TAIL_BROWS)],
                            xbuf.at[pl.ds(0, TAIL_BROWS)])
            pltpu.sync_copy(yf.at[pl.ds(toff, TAIL_BROWS)],
                            ybuf.at[pl.ds(0, TAIL_BROWS)])
            for t in range(9):
                obuf[pl.ds(t * L, L)] = compute_vec(wid * 9 + t)
            pltpu.sync_copy(obuf.at[pl.ds(0, 9 * L)],
                            out.at[pl.ds(MAIN_ROWS + wid * (9 * L), 9 * L)])

    return sc_kernel


_KERNEL = _make_kernel()


def kernel(x, y, alpha, beta, sigma, alpha_mask, AllSPL):
    params = jnp.concatenate([
        jnp.full((L,), beta, dtype=jnp.float32),
        jnp.full((L,), sigma, dtype=jnp.float32),
    ])
    return _KERNEL(
        x.reshape(BROWS, ROWW),
        y.reshape(BROWS, ROWW),
        alpha.reshape(-1),
        alpha_mask.reshape(-1),
        AllSPL.reshape(-1),
        params,
    )
